# Initial kernel scaffold; baseline (speedup 1.0000x reference)
import jax
import jax.numpy as jnp
from jax.experimental import pallas as pl


def kernel(z, coord, edge_index, n_nodes, atom_mask, edge_mask, dummy1, dummy2, embedding, layers, decoder):
    raise NotImplementedError("write your pallas kernel here")


# speedup vs baseline: 2.3416x; 2.3416x over previous
import functools

import jax
import jax.numpy as jnp
from jax import lax
from jax.experimental import pallas as pl
from jax.experimental.pallas import tpu as pltpu
from jax.experimental.pallas import tpu_sc as plsc

N = 10000
E = 320000
NHF = 128
NRBF = 50
NRP = 64
CUTOFF = 5.0
NW = 32
EW = E // NW
C = 128
NBLK = E // C
NPAD = 10240
RPT = NPAD // 16

_LN2 = 0.6931471805599453
_W = CUTOFF / (NRBF - 1)
_COEF = -0.5 / (_W * _W)


def _ssp(x):
    return jnp.maximum(x, 0.0) + jnp.log(1.0 + jnp.exp(-jnp.abs(x))) - _LN2


def _make_dist():
    mesh = plsc.VectorSubcoreMesh(core_axis_name="c", subcore_axis_name="s")

    @functools.partial(
        pl.kernel,
        mesh=mesh,
        out_type=jax.ShapeDtypeStruct((E,), jnp.float32),
        compiler_params=pltpu.CompilerParams(needs_layout_passes=False),
        scratch_types=[
            pltpu.VMEM((N,), jnp.float32),
            pltpu.VMEM((N,), jnp.float32),
            pltpu.VMEM((N,), jnp.float32),
            pltpu.VMEM((EW,), jnp.int32),
            pltpu.VMEM((EW,), jnp.int32),
            pltpu.VMEM((EW,), jnp.float32),
        ],
    )
    def dist_kernel(cx_hbm, cy_hbm, cz_hbm, row_hbm, col_hbm, d2_hbm,
                    cx, cy, cz, ri, ci, dv):
        cid = lax.axis_index("c")
        sid = lax.axis_index("s")
        wid = cid * 16 + sid
        base = pl.multiple_of(wid * EW, 8)
        pltpu.sync_copy(cx_hbm, cx)
        pltpu.sync_copy(cy_hbm, cy)
        pltpu.sync_copy(cz_hbm, cz)
        pltpu.sync_copy(row_hbm.at[pl.ds(base, EW)], ri)
        pltpu.sync_copy(col_hbm.at[pl.ds(base, EW)], ci)

        def body(i, carry):
            o = i * 16
            r = ri[pl.ds(o, 16)]
            c = ci[pl.ds(o, 16)]
            dx = plsc.load_gather(cx, [r]) - plsc.load_gather(cx, [c])
            dy = plsc.load_gather(cy, [r]) - plsc.load_gather(cy, [c])
            dz = plsc.load_gather(cz, [r]) - plsc.load_gather(cz, [c])
            dv[pl.ds(o, 16)] = dx * dx + dy * dy + dz * dz
            return carry

        lax.fori_loop(0, EW // 16, body, 0)
        pltpu.sync_copy(dv, d2_hbm.at[pl.ds(base, EW)])

    return dist_kernel


def _make_message():
    mesh = plsc.VectorSubcoreMesh(core_axis_name="c", subcore_axis_name="s")

    @functools.partial(
        pl.kernel,
        mesh=mesh,
        out_type=jax.ShapeDtypeStruct((2, NPAD, NHF), jnp.float32),
        compiler_params=pltpu.CompilerParams(needs_layout_passes=False),
        scratch_types=[
            pltpu.VMEM_SHARED((NPAD, NHF), jnp.float32),
            pltpu.VMEM((1, C), jnp.int32),
            pltpu.VMEM((C,), jnp.int32),
            pltpu.VMEM((C, NHF), jnp.float32),
            pltpu.VMEM((C, NHF), jnp.float32),
            pltpu.SemaphoreType.DMA,
        ],
    )
    def msg_kernel(xf_hbm, wij_hbm, row_hbm, col_hbm, out_hbm,
                   acc, rowv, colv, gath, wv, sem):
        cid = lax.axis_index("c")
        sid = lax.axis_index("s")
        wid = cid * 16 + sid

        def zbody(i, carry):
            for k in range(NHF // 16):
                wv[i, pl.ds(k * 16, 16)] = jnp.zeros((16,), jnp.float32)
            return carry

        lax.fori_loop(0, C, zbody, 0)
        r0 = pl.multiple_of(sid * RPT, 128)
        for j in range(RPT // C):
            pltpu.sync_copy(wv, acc.at[pl.ds(r0 + j * C, C)])
        plsc.subcore_barrier()

        nfull = NBLK // NW
        nblk = nfull + (wid < NBLK - nfull * NW).astype(jnp.int32)

        def chunk(j, carry):
            base = pl.multiple_of((j * NW + wid) * C, 8)
            pltpu.sync_copy(row_hbm.at[pl.ds(base, C)], rowv.at[0])
            pltpu.sync_copy(col_hbm.at[pl.ds(base, C)], colv)
            pltpu.async_copy(xf_hbm.at[colv], gath, sem).wait()
            pltpu.sync_copy(wij_hbm.at[pl.ds(base, C)], wv)

            def mbody(r, mc):
                for k in range(NHF // 16):
                    sl = pl.ds(k * 16, 16)
                    gath[r, sl] = gath[r, sl] * wv[r, sl]
                return mc

            lax.fori_loop(0, C, mbody, 0)
            pltpu.sync_copy(gath, acc.at[rowv.at[0]], add=True)
            return carry

        lax.fori_loop(0, nblk, chunk, 0)
        plsc.subcore_barrier()

        for j in range(RPT // C):
            pltpu.sync_copy(acc.at[pl.ds(r0 + j * C, C)],
                            out_hbm.at[cid, pl.ds(r0 + j * C, C)])

    return msg_kernel


_dist = _make_dist()
_message = _make_message()


BN = 1000
BE = 2560


def _emb_body(z_ref, emb_ref, w0_ref, h_ref, xf_ref):
    z = z_ref[...]
    iot = lax.broadcasted_iota(jnp.int32, (BN, 128), 1)
    onehot = (iot == z).astype(jnp.float32)
    h = jnp.dot(onehot, emb_ref[...], preferred_element_type=jnp.float32)
    h_ref[...] = h
    xf_ref[...] = jnp.dot(h, w0_ref[...], preferred_element_type=jnp.float32)


def _emb_in2f(z2, embp, w0):
    return pl.pallas_call(
        _emb_body,
        grid=(N // BN,),
        in_specs=[
            pl.BlockSpec((BN, 1), lambda i: (i, 0)),
            pl.BlockSpec((128, NHF), lambda i: (0, 0)),
            pl.BlockSpec((NHF, NHF), lambda i: (0, 0)),
        ],
        out_specs=[
            pl.BlockSpec((BN, NHF), lambda i: (i, 0)),
            pl.BlockSpec((BN, NHF), lambda i: (i, 0)),
        ],
        out_shape=[
            jax.ShapeDtypeStruct((N, NHF), jnp.float32),
            jax.ShapeDtypeStruct((N, NHF), jnp.float32),
        ],
    )(z2, embp, w0)


def _filter_body(d2_ref, em_ref, wf1_ref, bf1_ref, wf2_ref, bf2_ref, wij_ref):
    d = jnp.sqrt(d2_ref[...])
    offs = lax.broadcasted_iota(jnp.int32, (BE, NRP), 1).astype(jnp.float32) * _W
    rbf = jnp.exp(_COEF * (d - offs) ** 2)
    g = _ssp(jnp.dot(rbf, wf1_ref[...], preferred_element_type=jnp.float32)
             + bf1_ref[...])
    w = jnp.dot(g, wf2_ref[...], preferred_element_type=jnp.float32) + bf2_ref[...]
    cut = 0.5 * (jnp.cos(d * (jnp.pi / CUTOFF)) + 1.0)
    cut = cut * (d < CUTOFF).astype(jnp.float32) * em_ref[...]
    wij_ref[...] = w * cut


def _filter(d2, emask, wf1p, bf1, wf2, bf2):
    return pl.pallas_call(
        _filter_body,
        grid=(E // BE,),
        in_specs=[
            pl.BlockSpec((BE, 1), lambda i: (i, 0)),
            pl.BlockSpec((BE, 1), lambda i: (i, 0)),
            pl.BlockSpec((NRP, NHF), lambda i: (0, 0)),
            pl.BlockSpec((1, NHF), lambda i: (0, 0)),
            pl.BlockSpec((NHF, NHF), lambda i: (0, 0)),
            pl.BlockSpec((1, NHF), lambda i: (0, 0)),
        ],
        out_specs=pl.BlockSpec((BE, NHF), lambda i: (i, 0)),
        out_shape=jax.ShapeDtypeStruct((E, NHF), jnp.float32),
    )(d2, emask, wf1p, bf1, wf2, bf2)


def _agg_body(ms_ref, h_ref, wo1_ref, bo1_ref, wo2_ref, bo2_ref, wn_ref,
              h_out, xf_out):
    m = ms_ref[0] + ms_ref[1]
    t = _ssp(jnp.dot(m, wo1_ref[...], preferred_element_type=jnp.float32)
             + bo1_ref[...])
    m2 = jnp.dot(t, wo2_ref[...], preferred_element_type=jnp.float32) + bo2_ref[...]
    hn = h_ref[...] + m2
    h_out[...] = hn
    xf_out[...] = jnp.dot(hn, wn_ref[...], preferred_element_type=jnp.float32)


def _agg(ms, h, wo1, bo1, wo2, bo2, wnext):
    return pl.pallas_call(
        _agg_body,
        grid=(N // BN,),
        in_specs=[
            pl.BlockSpec((2, BN, NHF), lambda i: (0, i, 0)),
            pl.BlockSpec((BN, NHF), lambda i: (i, 0)),
            pl.BlockSpec((NHF, NHF), lambda i: (0, 0)),
            pl.BlockSpec((1, NHF), lambda i: (0, 0)),
            pl.BlockSpec((NHF, NHF), lambda i: (0, 0)),
            pl.BlockSpec((1, NHF), lambda i: (0, 0)),
            pl.BlockSpec((NHF, NHF), lambda i: (0, 0)),
        ],
        out_specs=[
            pl.BlockSpec((BN, NHF), lambda i: (i, 0)),
            pl.BlockSpec((BN, NHF), lambda i: (i, 0)),
        ],
        out_shape=[
            jax.ShapeDtypeStruct((N, NHF), jnp.float32),
            jax.ShapeDtypeStruct((N, NHF), jnp.float32),
        ],
    )(ms, h, wo1, bo1, wo2, bo2, wnext)


def _agg_pool_body(ms_ref, h_ref, wo1_ref, bo1_ref, wo2_ref, bo2_ref,
                   am_ref, hs_out):
    m = ms_ref[0] + ms_ref[1]
    t = _ssp(jnp.dot(m, wo1_ref[...], preferred_element_type=jnp.float32)
             + bo1_ref[...])
    m2 = jnp.dot(t, wo2_ref[...], preferred_element_type=jnp.float32) + bo2_ref[...]
    hn = (h_ref[...] + m2) * am_ref[...]
    blk = jnp.sum(hn, axis=0, keepdims=True)
    i = pl.program_id(0)

    @pl.when(i == 0)
    def _():
        hs_out[...] = blk

    @pl.when(i > 0)
    def _():
        hs_out[...] += blk


def _agg_pool(ms, h, wo1, bo1, wo2, bo2, amask):
    return pl.pallas_call(
        _agg_pool_body,
        grid=(N // BN,),
        in_specs=[
            pl.BlockSpec((2, BN, NHF), lambda i: (0, i, 0)),
            pl.BlockSpec((BN, NHF), lambda i: (i, 0)),
            pl.BlockSpec((NHF, NHF), lambda i: (0, 0)),
            pl.BlockSpec((1, NHF), lambda i: (0, 0)),
            pl.BlockSpec((NHF, NHF), lambda i: (0, 0)),
            pl.BlockSpec((1, NHF), lambda i: (0, 0)),
            pl.BlockSpec((BN, 1), lambda i: (i, 0)),
        ],
        out_specs=pl.BlockSpec((1, NHF), lambda i: (0, 0)),
        out_shape=jax.ShapeDtypeStruct((1, NHF), jnp.float32),
    )(ms, h, wo1, bo1, wo2, bo2, amask)


def _decoder_body(hs_ref, nn_ref, w1_ref, b1_ref, w2_ref, b2_ref, p_out):
    x = hs_ref[...] + float(N) * nn_ref[...]
    t = _ssp(jnp.dot(x, w1_ref[...], preferred_element_type=jnp.float32)
             + b1_ref[...])
    p_out[...] = jnp.dot(t, w2_ref[...], preferred_element_type=jnp.float32) \
        + b2_ref[...]


def _decoder(hs, nn, w1, b1, w2, b2):
    return pl.pallas_call(
        _decoder_body,
        grid=(1,),
        in_specs=[
            pl.BlockSpec((1, NHF), lambda i: (0, 0)),
            pl.BlockSpec((1, 1), lambda i: (0, 0)),
            pl.BlockSpec((NHF, NHF // 2), lambda i: (0, 0)),
            pl.BlockSpec((1, NHF // 2), lambda i: (0, 0)),
            pl.BlockSpec((NHF // 2, 1), lambda i: (0, 0)),
            pl.BlockSpec((1, 1), lambda i: (0, 0)),
        ],
        out_specs=pl.BlockSpec((1, 1), lambda i: (0, 0)),
        out_shape=jax.ShapeDtypeStruct((1, 1), jnp.float32),
    )(hs, nn, w1, b1, w2, b2)


def kernel(z, coord, edge_index, n_nodes, atom_mask, edge_mask, dummy1,
           dummy2, embedding, layers, decoder):
    row = edge_index[0].astype(jnp.int32)
    col = edge_index[1].astype(jnp.int32)
    c3 = coord.T.astype(jnp.float32)

    d2 = _dist(c3[0], c3[1], c3[2], row, col).reshape(E, 1)

    embp = jnp.zeros((128, NHF), jnp.float32).at[:embedding.shape[0]].set(embedding)
    z2 = z.astype(jnp.int32).reshape(N, 1)
    h, xf = _emb_in2f(z2, embp, layers[0]['W_in2f'])

    delta = (jnp.asarray(n_nodes, jnp.float32) - float(N)).reshape(1, 1)

    hs = None
    for li in range(len(layers)):
        p = layers[li]
        wf1p = jnp.zeros((NRP, NHF), jnp.float32).at[:NRBF].set(p['Wf1'])
        wij = _filter(d2, edge_mask, wf1p, p['bf1'].reshape(1, NHF),
                      p['Wf2'], p['bf2'].reshape(1, NHF))
        ms = _message(xf, wij, row, col)
        if li + 1 < len(layers):
            h, xf = _agg(ms, h, p['Wo1'], p['bo1'].reshape(1, NHF),
                         p['Wo2'], p['bo2'].reshape(1, NHF),
                         layers[li + 1]['W_in2f'])
        else:
            hs = _agg_pool(ms, h, p['Wo1'], p['bo1'].reshape(1, NHF),
                           p['Wo2'], p['bo2'].reshape(1, NHF), atom_mask)
    pred = _decoder(hs, delta, decoder['W1'],
                    decoder['b1'].reshape(1, NHF // 2), decoder['W2'],
                    decoder['b2'].reshape(1, 1))
    return pred.reshape(1)
